# baseline (speedup 1.0000x reference)
import functools

import jax
import jax.numpy as jnp
from jax import lax
from jax.experimental import pallas as pl
from jax.experimental.pallas import tpu as pltpu
from jax.experimental.pallas import tpu_sc as plsc

_NC = 2
_NS = 16
_NW = _NC * _NS


def _vq_idx_block(x_ref, e_ref, idx_ref, xc_ref, ec_ref, et_ref, esq_ref):
    x = x_ref[:]
    e = e_ref[:]

    @pl.when(pl.program_id(0) == 0)
    def _():
        esq_ref[:] = jnp.sum(e * e, axis=0, keepdims=True)
        ec_ref[:] = e
        et_ref[:] = e.T

    x_sq = jnp.sum(x * x, axis=1, keepdims=True)
    mm2 = jnp.dot(x + x, e, preferred_element_type=jnp.float32)
    d = (x_sq - mm2) + esq_ref[:]
    m = jnp.min(d, axis=1, keepdims=True)
    iota = jax.lax.broadcasted_iota(jnp.int32, (1, d.shape[1]), 1)
    n_embed = d.shape[1]
    idx = jnp.min(jnp.where(d == m, iota, n_embed), axis=1, keepdims=True)
    idx_ref[:] = idx.reshape(1, x.shape[0] // 128, 128)
    xc_ref[:] = x


def _sc_gather(et_hbm, idx_hbm, q_hbm, g_hbm, idx_v, rows_v, sem):
    wid = lax.axis_index("s") * _NC + lax.axis_index("c")
    b_per_w = idx_hbm.shape[0] // _NW
    base = wid * b_per_w
    pltpu.sync_copy(idx_hbm.at[pl.ds(base, b_per_w)], idx_v)
    pltpu.async_copy(et_hbm.at[idx_v], rows_v, sem).wait()
    pltpu.sync_copy(rows_v, q_hbm.at[pl.ds(base, b_per_w)])
    pltpu.sync_copy(rows_v, g_hbm.at[pl.ds(base, b_per_w)])


def kernel(input, embed):
    e_dim, n_embed = embed.shape
    flatten = input.reshape(-1, e_dim)
    n = flatten.shape[0]
    block_r = 2304
    grid = (n // block_r,)

    idx, xc, ec, et = pl.pallas_call(
        _vq_idx_block,
        grid=grid,
        in_specs=[
            pl.BlockSpec((block_r, e_dim), lambda i: (i, 0)),
            pl.BlockSpec((e_dim, n_embed), lambda i: (0, 0)),
        ],
        out_specs=[
            pl.BlockSpec((1, block_r // 128, 128), lambda i: (i, 0, 0)),
            pl.BlockSpec((block_r, e_dim), lambda i: (i, 0)),
            pl.BlockSpec((e_dim, n_embed), lambda i: (0, 0)),
            pl.BlockSpec((n_embed, e_dim), lambda i: (0, 0)),
        ],
        out_shape=[
            jax.ShapeDtypeStruct((n // block_r, block_r // 128, 128), jnp.int32),
            jax.ShapeDtypeStruct((n, e_dim), jnp.float32),
            jax.ShapeDtypeStruct((e_dim, n_embed), jnp.float32),
            jax.ShapeDtypeStruct((n_embed, e_dim), jnp.float32),
        ],
        scratch_shapes=[pltpu.VMEM((1, n_embed), jnp.float32)],
    )(flatten, embed)

    b_per_w = n // _NW
    sc = functools.partial(
        pl.kernel,
        mesh=plsc.VectorSubcoreMesh(core_axis_name="c", subcore_axis_name="s"),
        out_type=[
            jax.ShapeDtypeStruct((n, e_dim), jnp.float32),
            jax.ShapeDtypeStruct((n, e_dim), jnp.float32),
        ],
        scratch_types=[
            pltpu.VMEM((b_per_w,), jnp.int32),
            pltpu.VMEM((b_per_w, e_dim), jnp.float32),
            pltpu.SemaphoreType.DMA,
        ],
        compiler_params=pltpu.CompilerParams(use_tc_tiling_on_sc=False),
    )
    q, g = sc(_sc_gather)(et, idx.reshape(n))

    quantize = q.reshape(input.shape)
    embed_idxs = idx.reshape(input.shape[:-1])
    quantize_input_grad = g.reshape(input.shape)
    return (quantize, xc.reshape(input.shape), quantize_input_grad,
            embed_idxs, ec)


# speedup vs baseline: 1.3550x; 1.3550x over previous
import jax
import jax.numpy as jnp
from jax.experimental import pallas as pl
from jax.experimental.pallas import tpu as pltpu


def _vq_block(x_ref, e_ref, q_ref, g_ref, idx_ref, xc_ref, ec_ref, esq_ref):
    x = x_ref[:]
    e = e_ref[:]

    @pl.when(pl.program_id(0) == 0)
    def _():
        esq_ref[:] = jnp.sum(e * e, axis=0, keepdims=True)
        ec_ref[:] = e

    x_sq = jnp.sum(x * x, axis=1, keepdims=True)
    mm2 = jnp.dot(x + x, e, preferred_element_type=jnp.float32)
    d = (x_sq - mm2) + esq_ref[:]
    m = jnp.min(d, axis=1, keepdims=True)
    iota = jax.lax.broadcasted_iota(jnp.int32, (1, d.shape[1]), 1)
    n_embed = d.shape[1]
    idx = jnp.min(jnp.where(d == m, iota, n_embed), axis=1, keepdims=True)
    onehot = (iota == idx).astype(jnp.float32)
    q = jax.lax.dot_general(
        onehot, e, (((1,), (1,)), ((), ())),
        preferred_element_type=jnp.float32)
    q_ref[:] = q
    g_ref[:] = x + (q - x)
    idx_ref[:] = idx.reshape(1, x.shape[0] // 128, 128)
    xc_ref[:] = x


def kernel(input, embed):
    e_dim, n_embed = embed.shape
    flatten = input.reshape(-1, e_dim)
    n = flatten.shape[0]
    block_r = 2304
    grid = (n // block_r,)

    q, g, idx, xc, ec = pl.pallas_call(
        _vq_block,
        grid=grid,
        in_specs=[
            pl.BlockSpec((block_r, e_dim), lambda i: (i, 0)),
            pl.BlockSpec((e_dim, n_embed), lambda i: (0, 0)),
        ],
        out_specs=[
            pl.BlockSpec((block_r, e_dim), lambda i: (i, 0)),
            pl.BlockSpec((block_r, e_dim), lambda i: (i, 0)),
            pl.BlockSpec((1, block_r // 128, 128), lambda i: (i, 0, 0)),
            pl.BlockSpec((block_r, e_dim), lambda i: (i, 0)),
            pl.BlockSpec((e_dim, n_embed), lambda i: (0, 0)),
        ],
        out_shape=[
            jax.ShapeDtypeStruct((n, e_dim), jnp.float32),
            jax.ShapeDtypeStruct((n, e_dim), jnp.float32),
            jax.ShapeDtypeStruct((n // block_r, block_r // 128, 128), jnp.int32),
            jax.ShapeDtypeStruct((n, e_dim), jnp.float32),
            jax.ShapeDtypeStruct((e_dim, n_embed), jnp.float32),
        ],
        scratch_shapes=[pltpu.VMEM((1, n_embed), jnp.float32)],
    )(flatten, embed)

    quantize = q.reshape(input.shape)
    embed_idxs = idx.reshape(input.shape[:-1])
    quantize_input_grad = g.reshape(input.shape)
    return (quantize, xc.reshape(input.shape), quantize_input_grad,
            embed_idxs, ec)


# speedup vs baseline: 1.3648x; 1.0072x over previous
import jax
import jax.numpy as jnp
from jax.experimental import pallas as pl
from jax.experimental.pallas import tpu as pltpu


def _vq_block(x_ref, e_ref, q_ref, g_ref, idx_ref, xc_ref, ec_ref, esq_ref):
    x = x_ref[:]
    e = e_ref[:]

    @pl.when(pl.program_id(0) == 0)
    def _():
        esq_ref[:] = jnp.sum(e * e, axis=0, keepdims=True)
        ec_ref[:] = e

    x_sq = jnp.sum(x * x, axis=1, keepdims=True)
    mm2 = jnp.dot(x + x, e, preferred_element_type=jnp.float32)
    d = (x_sq - mm2) + esq_ref[:]
    m = jnp.min(d, axis=1, keepdims=True)
    iota = jax.lax.broadcasted_iota(jnp.int32, (1, d.shape[1]), 1)
    n_embed = d.shape[1]
    idx = jnp.min(jnp.where(d == m, iota, n_embed), axis=1, keepdims=True)
    onehot = (iota == idx).astype(jnp.float32)
    q = jax.lax.dot_general(
        onehot, e, (((1,), (1,)), ((), ())),
        preferred_element_type=jnp.float32)
    q_ref[:] = q
    g_ref[:] = x + (q - x)
    idx_ref[:] = idx.reshape(1, x.shape[0] // 128, 128)
    xc_ref[:] = x


def kernel(input, embed):
    e_dim, n_embed = embed.shape
    flatten = input.reshape(-1, e_dim)
    n = flatten.shape[0]
    block_r = 3072
    grid = (n // block_r,)

    q, g, idx, xc, ec = pl.pallas_call(
        _vq_block,
        grid=grid,
        in_specs=[
            pl.BlockSpec((block_r, e_dim), lambda i: (i, 0)),
            pl.BlockSpec((e_dim, n_embed), lambda i: (0, 0)),
        ],
        out_specs=[
            pl.BlockSpec((block_r, e_dim), lambda i: (i, 0)),
            pl.BlockSpec((block_r, e_dim), lambda i: (i, 0)),
            pl.BlockSpec((1, block_r // 128, 128), lambda i: (i, 0, 0)),
            pl.BlockSpec((block_r, e_dim), lambda i: (i, 0)),
            pl.BlockSpec((e_dim, n_embed), lambda i: (0, 0)),
        ],
        out_shape=[
            jax.ShapeDtypeStruct((n, e_dim), jnp.float32),
            jax.ShapeDtypeStruct((n, e_dim), jnp.float32),
            jax.ShapeDtypeStruct((n // block_r, block_r // 128, 128), jnp.int32),
            jax.ShapeDtypeStruct((n, e_dim), jnp.float32),
            jax.ShapeDtypeStruct((e_dim, n_embed), jnp.float32),
        ],
        scratch_shapes=[pltpu.VMEM((1, n_embed), jnp.float32)],
    )(flatten, embed)

    quantize = q.reshape(input.shape)
    embed_idxs = idx.reshape(input.shape[:-1])
    quantize_input_grad = g.reshape(input.shape)
    return (quantize, xc.reshape(input.shape), quantize_input_grad,
            embed_idxs, ec)
